# baseline (speedup 1.0000x reference)
import math

import jax
import jax.numpy as jnp
from jax import lax
from jax.experimental import pallas as pl
from jax.experimental.pallas import tpu as pltpu
from jax.experimental.pallas import tpu_sc as plsc

_B, _L, _D = 4096, 200, 64
_BT = _B * _L
_NC, _NS, _NL = 2, 16, 16
_NW = _NC * _NS
_TPW = _BT // _NW
_GROUP = 256
_NG = _TPW // _GROUP
_IDXROWS = _TPW // 128


def _prep_body(xm_ref, xy_ref, m_ref, y_ref, idx_ref, tab_ref):
    two_pi = 2.0 * math.pi
    am = two_pi / 12.0 * m_ref[...]
    ay = two_pi / 3.0 * y_ref[...]
    fm = jnp.sin(am) + jnp.cos(am)
    fy = jnp.sin(ay) + jnp.cos(ay)
    tab = jnp.concatenate([fm + fy[0:1], fm + fy[1:2], fm + fy[2:3]], axis=0)
    tab_ref[...] = jnp.pad(tab, ((0, 4), (0, 128 - _D)))
    yc = jnp.clip(xy_ref[...] - 22, 0, 2)
    idx_ref[...] = xm_ref[...] + 12 * yc


def _prep(xm, xy, month_embed, year_embed):
    return pl.pallas_call(
        _prep_body,
        out_shape=(
            jax.ShapeDtypeStruct((_BT // 128, 128), jnp.int32),
            jax.ShapeDtypeStruct((40, 128), jnp.float32),
        ),
    )(xm, xy, month_embed, year_embed)


def _gather_body(tab_hbm, idx_hbm, out_hbm, tab_v, idx_v, rows0, rows1, ssem):
    wid = lax.axis_index("s") * _NC + lax.axis_index("c")
    base = wid * _TPW
    pltpu.sync_copy(tab_hbm, tab_v)
    pltpu.sync_copy(idx_hbm.at[wid], idx_v)
    lanes = lax.iota(jnp.int32, _NL)
    cols = [lanes + k * _NL for k in range(_D // _NL)]
    picks = [jnp.full((_NL, 1), t3, jnp.int32) for t3 in range(_NL)]
    _dnums = lax.GatherDimensionNumbers(
        offset_dims=(), collapsed_slice_dims=(0,), start_index_map=(0,)
    )

    def lane_broadcast(vec, pick):
        return lax.gather(
            vec,
            pick,
            dimension_numbers=_dnums,
            slice_sizes=(1,),
            mode=lax.GatherScatterMode.PROMISE_IN_BOUNDS,
        )

    def compute_group(g, rows_ref):
        def row_body(r4, carry):
            rowsplat = jnp.full((_NL,), g * 2 + r4, jnp.int32)
            for t2 in range(8):
                idx16 = plsc.load_gather(idx_v, [rowsplat, lanes + t2 * _NL])
                for t3 in range(0, _NL, 4):
                    vals = []
                    for u in range(4):
                        trow = lane_broadcast(idx16, picks[t3 + u])
                        vals.extend(
                            plsc.load_gather(tab_v, [trow, cols[k]])
                            for k in range(_D // _NL)
                        )
                    for u in range(4):
                        outrow = jnp.full(
                            (_NL,), r4 * 128 + t2 * _NL + t3 + u, jnp.int32
                        )
                        for k in range(_D // _NL):
                            plsc.store_scatter(
                                rows_ref,
                                [outrow, cols[k]],
                                vals[u * (_D // _NL) + k],
                            )
            return carry

        lax.fori_loop(0, 2, row_body, 0)

    def do_group(gg, parity, rows_ref):
        g = gg * 2 + parity

        @pl.when(gg >= 1)
        def _():
            pltpu.make_async_copy(
                rows_ref, out_hbm.at[pl.ds(0, _GROUP)], ssem
            ).wait()

        compute_group(g, rows_ref)
        pltpu.async_copy(
            rows_ref, out_hbm.at[pl.ds(base + g * _GROUP, _GROUP)], ssem
        )

    def body(gg, carry):
        do_group(gg, 0, rows0)
        do_group(gg, 1, rows1)
        return carry

    lax.fori_loop(0, _NG // 2, body, 0)
    for rows_ref in (rows0, rows1):
        pltpu.make_async_copy(
            rows_ref, out_hbm.at[pl.ds(0, _GROUP)], ssem
        ).wait()


_gather = pl.kernel(
    _gather_body,
    out_type=jax.ShapeDtypeStruct((_BT, _D), jnp.float32),
    mesh=plsc.VectorSubcoreMesh(core_axis_name="c", subcore_axis_name="s"),
    scratch_types=[
        pltpu.VMEM((40, 128), jnp.float32),
        pltpu.VMEM((_IDXROWS, 128), jnp.int32),
        pltpu.VMEM((_GROUP, _D), jnp.float32),
        pltpu.VMEM((_GROUP, _D), jnp.float32),
        pltpu.SemaphoreType.DMA,
    ],
    compiler_params=pltpu.CompilerParams(
        use_tc_tiling_on_sc=True, needs_layout_passes=False
    ),
)


def kernel(x, month_embed, year_embed):
    xm = x[..., 0].reshape(_BT // 128, 128)
    xy = x[..., 1].reshape(_BT // 128, 128)
    idx, tab = _prep(xm, xy, month_embed, year_embed)
    out = _gather(tab, idx.reshape(_NW, _IDXROWS, 128))
    return out.reshape(_B, _L, _D)


# speedup vs baseline: 1.0119x; 1.0119x over previous
import math

import jax
import jax.numpy as jnp
from jax import lax
from jax.experimental import pallas as pl
from jax.experimental.pallas import tpu as pltpu
from jax.experimental.pallas import tpu_sc as plsc

_B, _L, _D = 4096, 200, 64
_BT = _B * _L
_NC, _NS, _NL = 2, 16, 16
_NW = _NC * _NS
_TPW = _BT // _NW
_GROUP = 256
_NG = _TPW // _GROUP
_IDXROWS = _TPW // 128


def _prep_body(xm_ref, xy_ref, m_ref, y_ref, idx_ref, tab_ref):
    two_pi = 2.0 * math.pi
    am = two_pi / 12.0 * m_ref[...]
    ay = two_pi / 3.0 * y_ref[...]
    fm = jnp.sin(am) + jnp.cos(am)
    fy = jnp.sin(ay) + jnp.cos(ay)
    tab = jnp.concatenate([fm + fy[0:1], fm + fy[1:2], fm + fy[2:3]], axis=0)
    tab_ref[...] = jnp.pad(tab, ((0, 4), (0, 128 - _D)))
    yc = jnp.clip(xy_ref[...] - 22, 0, 2)
    idx_ref[...] = xm_ref[...] + 12 * yc


def _prep(xm, xy, month_embed, year_embed):
    return pl.pallas_call(
        _prep_body,
        out_shape=(
            jax.ShapeDtypeStruct((_BT // 128, 128), jnp.int32),
            jax.ShapeDtypeStruct((40, 128), jnp.float32),
        ),
    )(xm, xy, month_embed, year_embed)


def _gather_body(tab_hbm, idx_hbm, out_hbm, tab_v, idx_v, rows0, rows1, ssem):
    wid = lax.axis_index("s") * _NC + lax.axis_index("c")
    base = wid * _TPW
    pltpu.sync_copy(tab_hbm, tab_v)
    pltpu.sync_copy(idx_hbm.at[wid], idx_v)
    lanes = lax.iota(jnp.int32, _NL)
    cols = [lanes + k * _NL for k in range(_D // _NL)]
    picks = [jnp.full((_NL, 1), t3, jnp.int32) for t3 in range(_NL)]
    _dnums = lax.GatherDimensionNumbers(
        offset_dims=(), collapsed_slice_dims=(0,), start_index_map=(0,)
    )

    def lane_broadcast(vec, pick):
        return lax.gather(
            vec,
            pick,
            dimension_numbers=_dnums,
            slice_sizes=(1,),
            mode=lax.GatherScatterMode.PROMISE_IN_BOUNDS,
        )

    def compute_group(g, rows_ref):
        def row_body(r4, carry):
            rowsplat = jnp.full((_NL,), g * 2 + r4, jnp.int32)
            for t2 in range(8):
                idx16 = plsc.load_gather(idx_v, [rowsplat, lanes + t2 * _NL])
                for t3 in range(0, _NL, 2):
                    vals = []
                    for u in range(2):
                        trow = lane_broadcast(idx16, picks[t3 + u])
                        vals.extend(
                            plsc.load_gather(tab_v, [trow, cols[k]])
                            for k in range(_D // _NL)
                        )
                    for u in range(2):
                        outrow = jnp.full(
                            (_NL,), r4 * 128 + t2 * _NL + t3 + u, jnp.int32
                        )
                        for k in range(_D // _NL):
                            plsc.store_scatter(
                                rows_ref,
                                [outrow, cols[k]],
                                vals[u * (_D // _NL) + k],
                            )
            return carry

        lax.fori_loop(0, 2, row_body, 0)

    def do_group(gg, parity, rows_ref):
        g = gg * 2 + parity

        @pl.when(gg >= 1)
        def _():
            pltpu.make_async_copy(
                rows_ref, out_hbm.at[pl.ds(0, _GROUP)], ssem
            ).wait()

        compute_group(g, rows_ref)
        pltpu.async_copy(
            rows_ref, out_hbm.at[pl.ds(base + g * _GROUP, _GROUP)], ssem
        )

    def body(gg, carry):
        do_group(gg, 0, rows0)
        do_group(gg, 1, rows1)
        return carry

    lax.fori_loop(0, _NG // 2, body, 0)
    for rows_ref in (rows0, rows1):
        pltpu.make_async_copy(
            rows_ref, out_hbm.at[pl.ds(0, _GROUP)], ssem
        ).wait()


_gather = pl.kernel(
    _gather_body,
    out_type=jax.ShapeDtypeStruct((_BT, _D), jnp.float32),
    mesh=plsc.VectorSubcoreMesh(core_axis_name="c", subcore_axis_name="s"),
    scratch_types=[
        pltpu.VMEM((40, 128), jnp.float32),
        pltpu.VMEM((_IDXROWS, 128), jnp.int32),
        pltpu.VMEM((_GROUP, _D), jnp.float32),
        pltpu.VMEM((_GROUP, _D), jnp.float32),
        pltpu.SemaphoreType.DMA,
    ],
    compiler_params=pltpu.CompilerParams(
        use_tc_tiling_on_sc=True, needs_layout_passes=False
    ),
)


def kernel(x, month_embed, year_embed):
    xm = x[..., 0].reshape(_BT // 128, 128)
    xy = x[..., 1].reshape(_BT // 128, 128)
    idx, tab = _prep(xm, xy, month_embed, year_embed)
    out = _gather(tab, idx.reshape(_NW, _IDXROWS, 128))
    return out.reshape(_B, _L, _D)
